# baseline (speedup 1.0000x reference)
import jax
import jax.numpy as jnp
from jax import lax
from jax.experimental import pallas as pl
from jax.experimental.pallas import tpu as pltpu
from jax.experimental.pallas import tpu_sc as plsc

B, T, D = 2, 2048, 1024
NH, NS, HD, SPH, HM = 4, 64, 256, 16, 4096
K_TOTAL = 8
TTR = 1024
NTR = (B * T) // TTR
TT = 1024
NT = T // TT
HMS = HM // 2


def _gelu(v):
    return 0.5 * v * (1.0 + jax.lax.erf(v * 0.7071067811865476))


def _router_kernel(x_ref, w1a_ref, w1b_ref, s64_ref, b1_ref, w2_ref, b2_ref,
                   tau_ref, a_ref, b1e_ref):
    @pl.when(pl.program_id(0) == 0)
    def _():
        m = jnp.mean(s64_ref[...], axis=0, keepdims=True)
        smf = jnp.concatenate([m, m, m, m], axis=1)
        smf8 = jnp.broadcast_to(smf, (8, D))
        b1e_ref[...] = (
            jnp.dot(smf8, w1b_ref[...], preferred_element_type=jnp.float32)
            + b1_ref[...]
        )

    h = jnp.dot(x_ref[...], w1a_ref[...], preferred_element_type=jnp.float32)
    h = _gelu(h + b1e_ref[0:1])
    scale = 1.0 / (jnp.abs(tau_ref[0, 0]) + 0.1)
    a_ref[...] = (jnp.dot(h, w2_ref[...], preferred_element_type=jnp.float32)
                  + b2_ref[...]) * scale


CHUNK = (B * T) // 32


def _route_sc_body(lg_hbm, al_hbm, lg_v, al_v):
    wid = lax.axis_index("s") * 2 + lax.axis_index("c")
    base = wid * CHUNK
    pltpu.sync_copy(lg_hbm.at[pl.ds(base, CHUNK)], lg_v)
    i16 = lax.iota(jnp.int32, 16)
    zero16 = jnp.zeros((16,), jnp.float32)
    zi = jnp.zeros((16,), jnp.int32)
    neg = jnp.full((16,), -1e30, jnp.float32)

    def zrow(r, carry):
        al_v[r, 0:16] = zero16
        al_v[r, 16:32] = zero16
        al_v[r, 32:48] = zero16
        al_v[r, 48:64] = zero16
        return carry

    lax.fori_loop(0, CHUNK, zrow, 0)

    def group(g, carry):
        rows = i16 + g * 16

        def step(sl, st):
            vals = list(st[:K_TOTAL])
            idxs = list(st[K_TOTAL:])
            cur = plsc.load_gather(lg_v, [rows, zi + sl])
            curi = zi + sl
            for j in range(K_TOTAL):
                m = cur > vals[j]
                vj = jnp.where(m, cur, vals[j])
                cur = jnp.where(m, vals[j], cur)
                ij = jnp.where(m, curi, idxs[j])
                curi = jnp.where(m, idxs[j], curi)
                vals[j] = vj
                idxs[j] = ij
            return tuple(vals) + tuple(idxs)

        st = lax.fori_loop(0, NS, step, (neg,) * K_TOTAL + (zi,) * K_TOTAL)
        vals = st[:K_TOTAL]
        idxs = st[K_TOTAL:]
        es = [jnp.exp(v - vals[0]) for v in vals]
        den = es[0]
        for e in es[1:]:
            den = den + e
        inv = 1.0 / den
        for j in range(K_TOTAL):
            plsc.store_scatter(al_v, [rows, idxs[j]], es[j] * inv)
        return carry

    lax.fori_loop(0, CHUNK // 16, group, 0)
    pltpu.sync_copy(al_v, al_hbm.at[pl.ds(base, CHUNK)])


_route_sc = pl.kernel(
    _route_sc_body,
    out_type=jax.ShapeDtypeStruct((B * T, NS), jnp.float32),
    mesh=plsc.VectorSubcoreMesh(core_axis_name="c", subcore_axis_name="s"),
    scratch_types=[pltpu.VMEM((CHUNK, NS), jnp.float32),
                   pltpu.VMEM((CHUNK, NS), jnp.float32)],
    compiler_params=pltpu.CompilerParams(needs_layout_passes=False),
)


def _mid_kernel(a_ref, x_ref, s64_ref, wihT_ref, whhT_ref, bih_ref, bhh_ref,
                whp_ref, bhp_ref, woh_ref, boh_ref, sbig_ref):
    a = a_ref[0]
    xb = x_ref[0]
    dn = (((0,), (0,)), ((), ()))
    si = jax.lax.dot_general(a, xb, dn, preferred_element_type=jnp.float32)
    ones = jnp.ones((T, 8), jnp.float32)
    cs = jax.lax.dot_general(a, ones, dn, preferred_element_type=jnp.float32)
    si = si / (cs[:, 0:1] + 1e-8)

    s64 = s64_ref[...]
    blocks = [si[h * SPH:(h + 1) * SPH, h * HD:(h + 1) * HD]
              for h in range(NH)]
    sif = jnp.concatenate(blocks, axis=0)
    gi = (jnp.dot(sif, wihT_ref[...], preferred_element_type=jnp.float32)
          + bih_ref[...])
    gh = (jnp.dot(s64, whhT_ref[...], preferred_element_type=jnp.float32)
          + bhh_ref[...])
    r = jax.nn.sigmoid(gi[:, :HD] + gh[:, :HD])
    z = jax.nn.sigmoid(gi[:, HD:2 * HD] + gh[:, HD:2 * HD])
    n = jnp.tanh(gi[:, 2 * HD:] + r * gh[:, 2 * HD:])
    snew = (1.0 - z) * n + z * s64
    hmid = _gelu(jnp.dot(snew, whp_ref[...],
                         preferred_element_type=jnp.float32) + bhp_ref[...])
    snew = (jnp.dot(hmid, woh_ref[...], preferred_element_type=jnp.float32)
            + boh_ref[...])
    hblocks = []
    for h in range(NH):
        parts = []
        if h > 0:
            parts.append(jnp.zeros((SPH, h * HD), jnp.float32))
        parts.append(snew[h * SPH:(h + 1) * SPH])
        if h < NH - 1:
            parts.append(jnp.zeros((SPH, (NH - 1 - h) * HD), jnp.float32))
        hblocks.append(jnp.concatenate(parts, axis=1))
    sbig_ref[0] = jnp.concatenate(hblocks, axis=0)


def _out_kernel(a_ref, sb_ref, wvp_ref, bvp_ref, wvo_ref, bvo_ref,
                wop_ref, bop_ref, o_ref):
    bf16 = jnp.bfloat16
    u = jnp.dot(a_ref[0].astype(bf16), sb_ref[0],
                preferred_element_type=jnp.float32).astype(bf16)
    y = bvo_ref[...] * jnp.ones((TT, 1), jnp.float32)
    for j in range(HM // HMS):
        hj = _gelu(jnp.dot(u, wvp_ref[:, j * HMS:(j + 1) * HMS],
                           preferred_element_type=jnp.float32)
                   + bvp_ref[:, j * HMS:(j + 1) * HMS])
        y = y + jnp.dot(hj.astype(bf16), wvo_ref[j * HMS:(j + 1) * HMS, :],
                        preferred_element_type=jnp.float32)
    o_ref[0] = (jnp.dot(y.astype(bf16), wop_ref[...],
                        preferred_element_type=jnp.float32) + bop_ref[...])


def kernel(x, slot_init, W1, b1, W2, b2, Wih, Whh, bih, bhh, Whp, bhp,
           Woh, boh, Wvp, bvp, Wvo, bvo, Wop, bop, tau):
    f32 = jnp.float32
    bf16 = jnp.bfloat16
    xf = x.reshape(B * T, D)
    s64 = slot_init.reshape(NS, HD)
    W1a, W1b = W1[:D], W1[D:]

    logits = pl.pallas_call(
        _router_kernel,
        grid=(NTR,),
        in_specs=[
            pl.BlockSpec((TTR, D), lambda i: (i, 0)),
            pl.BlockSpec((D, D), lambda i: (0, 0)),
            pl.BlockSpec((D, D), lambda i: (0, 0)),
            pl.BlockSpec((NS, HD), lambda i: (0, 0)),
            pl.BlockSpec((1, D), lambda i: (0, 0)),
            pl.BlockSpec((D, NS), lambda i: (0, 0)),
            pl.BlockSpec((1, NS), lambda i: (0, 0)),
            pl.BlockSpec(memory_space=pltpu.SMEM),
        ],
        out_specs=pl.BlockSpec((TTR, NS), lambda i: (i, 0)),
        out_shape=jax.ShapeDtypeStruct((B * T, NS), f32),
        scratch_shapes=[pltpu.VMEM((8, D), f32)],
    )(xf, W1a, W1b, s64, b1.reshape(1, D), W2, b2.reshape(1, NS),
      tau.reshape(1, 1))

    alpha = _route_sc(logits)
    a3 = alpha.reshape(B, T, NS)
    _PROF = 0
    if _PROF == 1:
        return jnp.zeros((B, T, D), f32) + alpha[0, 0]
    x3 = xf.reshape(B, T, D)
    sbig = pl.pallas_call(
        _mid_kernel,
        grid=(B,),
        in_specs=[
            pl.BlockSpec((1, T, NS), lambda b: (b, 0, 0)),
            pl.BlockSpec((1, T, D), lambda b: (b, 0, 0)),
            pl.BlockSpec((NS, HD), lambda b: (0, 0)),
            pl.BlockSpec((HD, 3 * HD), lambda b: (0, 0)),
            pl.BlockSpec((HD, 3 * HD), lambda b: (0, 0)),
            pl.BlockSpec((1, 3 * HD), lambda b: (0, 0)),
            pl.BlockSpec((1, 3 * HD), lambda b: (0, 0)),
            pl.BlockSpec((HD, 4 * HD), lambda b: (0, 0)),
            pl.BlockSpec((1, 4 * HD), lambda b: (0, 0)),
            pl.BlockSpec((4 * HD, HD), lambda b: (0, 0)),
            pl.BlockSpec((1, HD), lambda b: (0, 0)),
        ],
        out_specs=pl.BlockSpec((1, NS, D), lambda b: (b, 0, 0)),
        out_shape=jax.ShapeDtypeStruct((B, NS, D), f32),
    )(a3, x3, s64, Wih.T, Whh.T, bih.reshape(1, 3 * HD),
      bhh.reshape(1, 3 * HD), Whp, bhp.reshape(1, 4 * HD), Woh,
      boh.reshape(1, HD))

    if _PROF == 2:
        return jnp.zeros((B, T, D), f32) + sbig[0, 0, 0]
    out = pl.pallas_call(
        _out_kernel,
        grid=(B, NT),
        in_specs=[
            pl.BlockSpec((1, TT, NS), lambda b, t: (b, t, 0)),
            pl.BlockSpec((1, NS, D), lambda b, t: (b, 0, 0)),
            pl.BlockSpec((D, HM), lambda b, t: (0, 0)),
            pl.BlockSpec((1, HM), lambda b, t: (0, 0)),
            pl.BlockSpec((HM, D), lambda b, t: (0, 0)),
            pl.BlockSpec((1, D), lambda b, t: (0, 0)),
            pl.BlockSpec((D, D), lambda b, t: (0, 0)),
            pl.BlockSpec((1, D), lambda b, t: (0, 0)),
        ],
        out_specs=pl.BlockSpec((1, TT, D), lambda b, t: (b, t, 0)),
        out_shape=jax.ShapeDtypeStruct((B, T, D), f32),
    )(a3, sbig.astype(bf16), Wvp.astype(bf16), bvp.reshape(1, HM),
      Wvo.astype(bf16), bvo.reshape(1, D),
      Wop.astype(bf16), bop.reshape(1, D))

    return out
